# baseline (speedup 1.0000x reference)
import jax
import jax.numpy as jnp
from jax.experimental import pallas as pl


_SEQ_BLOCK = 512


def _copy_kernel(pos_ref, out_ref):
    out_ref[0] = pos_ref[...]


def kernel(x, pos_embedding):
    batch, seq_len, head_dim = x.shape
    n_blocks = seq_len // _SEQ_BLOCK
    return pl.pallas_call(
        _copy_kernel,
        grid=(n_blocks, batch),
        in_specs=[pl.BlockSpec((_SEQ_BLOCK, head_dim), lambda s, b: (s, 0))],
        out_specs=pl.BlockSpec(
            (1, _SEQ_BLOCK, head_dim), lambda s, b: (b, s, 0)
        ),
        out_shape=jax.ShapeDtypeStruct(
            (batch, seq_len, head_dim), pos_embedding.dtype
        ),
    )(pos_embedding)


# speedup vs baseline: 1.4056x; 1.4056x over previous
import jax
import jax.numpy as jnp
from jax.experimental import pallas as pl


_SEQ_BLOCK = 512


def _bcast_kernel(pos_ref, out_ref):
    out_ref[...] = jnp.broadcast_to(pos_ref[...][None], out_ref.shape)


def kernel(x, pos_embedding):
    batch, seq_len, head_dim = x.shape
    n_blocks = seq_len // _SEQ_BLOCK
    return pl.pallas_call(
        _bcast_kernel,
        grid=(n_blocks,),
        in_specs=[pl.BlockSpec((_SEQ_BLOCK, head_dim), lambda s: (s, 0))],
        out_specs=pl.BlockSpec(
            (batch, _SEQ_BLOCK, head_dim), lambda s: (0, s, 0)
        ),
        out_shape=jax.ShapeDtypeStruct(
            (batch, seq_len, head_dim), pos_embedding.dtype
        ),
    )(pos_embedding)
